# baseline (speedup 1.0000x reference)
import functools

import jax
import jax.numpy as jnp
from jax import lax
from jax.experimental import pallas as pl
from jax.experimental.pallas import tpu as pltpu
from jax.experimental.pallas import tpu_sc as plsc

_N = 10000
_E = 160000
_D = 256
_HALF = 128
_NSUB = 16
_NCORE = 2
_EPT = _E // _NSUB
_CH = 128
_NCHUNK = 80
_EPAD = _NCHUNK * _CH
_W = 16
_NW = _NCHUNK // _W
_NP = 10112
_ZR = _NP // _NSUB
_R = 1000
_G = _N // _R


def _agg_body(zflat, src2, dstp, zeros, out, dst_b, sw0, sw1, g0, g1, sp,
              sem0, sem1, ssem0, ssem1, wsem):
    c = lax.axis_index("c")
    s = lax.axis_index("s")
    pltpu.sync_copy(dstp.at[s], dst_b)
    pltpu.sync_copy(src2.at[c, s, pl.ds(0, _W)], sw0)
    pltpu.sync_copy(zeros, sp.at[pl.ds(s * _ZR, _ZR)])
    plsc.subcore_barrier()

    for w in range(_NW):
        sw = (sw0, sw1)[w % 2]
        nx = (sw1, sw0)[w % 2]
        if w < _NW - 1:
            wcp = pltpu.async_copy(src2.at[c, s, pl.ds((w + 1) * _W, _W)],
                                   nx, wsem)
        base = w * _W
        pltpu.async_copy(zflat.at[sw.at[0]], g0, sem0)
        pltpu.async_copy(zflat.at[sw.at[1]], g1, sem1)

        def body(i, carry, sw=sw, base=base):
            j0 = 2 * i
            pltpu.make_async_copy(zflat.at[sw.at[j0]], g0, sem0).wait()
            pltpu.async_copy(g0, sp.at[dst_b.at[base + j0]], ssem0, add=True)
            pltpu.make_async_copy(zflat.at[sw.at[j0 + 1]], g1, sem1).wait()
            pltpu.async_copy(g1, sp.at[dst_b.at[base + j0 + 1]], ssem1,
                             add=True)
            pltpu.make_async_copy(g0, sp.at[dst_b.at[base + j0]],
                                  ssem0).wait()

            @pl.when(i < _W // 2 - 1)
            def _():
                pltpu.async_copy(zflat.at[sw.at[j0 + 2]], g0, sem0)

            pltpu.make_async_copy(g1, sp.at[dst_b.at[base + j0 + 1]],
                                  ssem1).wait()

            @pl.when(i < _W // 2 - 1)
            def _():
                pltpu.async_copy(zflat.at[sw.at[j0 + 3]], g1, sem1)

            return carry

        lax.fori_loop(0, _W // 2, body, 0)
        if w < _NW - 1:
            wcp.wait()
    plsc.subcore_barrier()
    pltpu.sync_copy(sp.at[pl.ds(s * _ZR, _ZR)],
                    out.at[pl.ds(c * _NP + s * _ZR, _ZR)])


@functools.lru_cache(maxsize=1)
def _make_aggregate():
    return functools.partial(
        pl.kernel,
        mesh=plsc.VectorSubcoreMesh(core_axis_name="c", subcore_axis_name="s"),
        out_type=jax.ShapeDtypeStruct((_NCORE * _NP, _HALF), jnp.float32),
        scratch_types=[
            pltpu.VMEM((_NCHUNK, _CH), jnp.int32),
            pltpu.VMEM((_W, _CH), jnp.int32),
            pltpu.VMEM((_W, _CH), jnp.int32),
            pltpu.VMEM((_CH, _HALF), jnp.float32),
            pltpu.VMEM((_CH, _HALF), jnp.float32),
            pltpu.VMEM_SHARED((_NP, _HALF), jnp.float32),
            pltpu.SemaphoreType.DMA,
            pltpu.SemaphoreType.DMA,
            pltpu.SemaphoreType.DMA,
            pltpu.SemaphoreType.DMA,
            pltpu.SemaphoreType.DMA,
        ],
    )(_agg_body)


def _aggregate(zflat, src2, dstp, zeros):
    return _make_aggregate()(zflat, src2, dstp, zeros)


def _mlp_body(z_ref, alo_ref, ahi_ref, w1_ref, b1_ref, w2_ref, b2_ref,
              h_ref, ps_ref, pss_ref):
    i = pl.program_id(0)
    s = z_ref[...] + jnp.concatenate([alo_ref[0], ahi_ref[0]], axis=1)
    dn = (((1,), (1,)), ((), ()))
    h1 = jnp.maximum(
        lax.dot_general(s, w1_ref[...], dn,
                        preferred_element_type=jnp.float32) + b1_ref[...], 0.0)
    h2 = jnp.maximum(
        lax.dot_general(h1, w2_ref[...], dn,
                        preferred_element_type=jnp.float32) + b2_ref[...], 0.0)
    h_ref[...] = h2

    @pl.when(i == 0)
    def _():
        ps_ref[...] = jnp.zeros_like(ps_ref)
        pss_ref[...] = jnp.zeros_like(pss_ref)

    ps_ref[0:1, :] += jnp.sum(h2, axis=0, keepdims=True)
    pss_ref[0:1, :] += jnp.sum(h2 * h2, axis=0, keepdims=True)


def _mlp(z, aggp, w1, b1, w2, b2):
    return pl.pallas_call(
        _mlp_body,
        grid=(_G,),
        in_specs=[
            pl.BlockSpec((_R, _D), lambda b: (b, 0)),
            pl.BlockSpec((1, _R, _HALF), lambda b: (0, b, 0)),
            pl.BlockSpec((1, _R, _HALF), lambda b: (1, b, 0)),
            pl.BlockSpec((_D, _D), lambda b: (0, 0)),
            pl.BlockSpec((1, _D), lambda b: (0, 0)),
            pl.BlockSpec((_D, _D), lambda b: (0, 0)),
            pl.BlockSpec((1, _D), lambda b: (0, 0)),
        ],
        out_specs=[
            pl.BlockSpec((_R, _D), lambda b: (b, 0)),
            pl.BlockSpec((8, _D), lambda b: (0, 0)),
            pl.BlockSpec((8, _D), lambda b: (0, 0)),
        ],
        out_shape=[
            jax.ShapeDtypeStruct((_N, _D), jnp.float32),
            jax.ShapeDtypeStruct((8, _D), jnp.float32),
            jax.ShapeDtypeStruct((8, _D), jnp.float32),
        ],
    )(z, aggp, aggp, w1, b1.reshape(1, _D), w2, b2.reshape(1, _D))


def _norm_body(h_ref, sc_ref, sh_ref, z_ref, zf_ref):
    hh = pl.program_id(1)
    sc = jnp.where(hh == 0, sc_ref[0:1, :], sc_ref[1:2, :])
    sh = jnp.where(hh == 0, sh_ref[0:1, :], sh_ref[1:2, :])
    v = h_ref[...] * sc + sh
    z_ref[...] = v
    zf_ref[0] = v


def _norm(h, scale, shift):
    return pl.pallas_call(
        _norm_body,
        grid=(_G, 2),
        in_specs=[
            pl.BlockSpec((_R, _HALF), lambda b, hh: (b, hh)),
            pl.BlockSpec((2, _HALF), lambda b, hh: (0, 0)),
            pl.BlockSpec((2, _HALF), lambda b, hh: (0, 0)),
        ],
        out_specs=[
            pl.BlockSpec((_R, _HALF), lambda b, hh: (b, hh)),
            pl.BlockSpec((1, _R, _HALF), lambda b, hh: (hh, b, 0)),
        ],
        out_shape=[
            jax.ShapeDtypeStruct((_N, _D), jnp.float32),
            jax.ShapeDtypeStruct((_NCORE, _NP, _HALF), jnp.float32),
        ],
    )(h, scale.reshape(2, _HALF), shift.reshape(2, _HALF))


def _layer(z, zflat, src2, dstp, zeros, w1, b1, w2, b2, gamma, beta):
    aggf = _aggregate(zflat, src2, dstp, zeros)
    h, ps, pss = _mlp(z, aggf.reshape(_NCORE, _NP, _HALF), w1, b1, w2, b2)
    mean = ps[0] / _N
    var = pss[0] / _N - mean * mean
    scale = gamma * lax.rsqrt(var + 1e-5)
    shift = beta - mean * scale
    z, zf = _norm(h, scale, shift)
    return z, zf.reshape(_NCORE * _NP, _HALF)


def _impl(x, edge_index, W1_0, b1_0, W2_0, b2_0, gamma_0, beta_0,
          W1_1, b1_1, W2_1, b2_1, gamma_1, beta_1):
    src = edge_index[0]
    dst = edge_index[1]
    srcr = jnp.pad(src.reshape(_NSUB, _EPT), ((0, 0), (0, _EPAD - _EPT)))
    srcr = srcr.reshape(_NSUB, _NCHUNK, _CH)
    src2 = jnp.stack([srcr, srcr + _NP])
    dstr = jnp.pad(dst.reshape(_NSUB, _EPT), ((0, 0), (0, _EPAD - _EPT)),
                   constant_values=_N)
    dstp = dstr.reshape(_NSUB, _NCHUNK, _CH)
    zeros = jnp.zeros((_ZR, _HALF), jnp.float32)

    xp = jnp.pad(x.reshape(_N, 2, _HALF).transpose(1, 0, 2),
                 ((0, 0), (0, _NP - _N), (0, 0)))
    xflat = xp.reshape(_NCORE * _NP, _HALF)
    z1, z1flat = _layer(x, xflat, src2, dstp, zeros,
                        W1_0, b1_0, W2_0, b2_0, gamma_0, beta_0)
    z2, _ = _layer(z1, z1flat, src2, dstp, zeros,
                   W1_1, b1_1, W2_1, b2_1, gamma_1, beta_1)
    return jnp.concatenate([z1, z2], axis=1)


kernel = jax.jit(_impl)


# speedup vs baseline: 1.1224x; 1.1224x over previous
import functools

import jax
import jax.numpy as jnp
from jax import lax
from jax.experimental import pallas as pl
from jax.experimental.pallas import tpu as pltpu
from jax.experimental.pallas import tpu_sc as plsc

_N = 10000
_E = 160000
_D = 256
_HALF = 128
_NSUB = 16
_NCORE = 2
_EPT = _E // _NSUB
_CH = 128
_NCHUNK = 80
_EPAD = _NCHUNK * _CH
_W = 16
_NW = _NCHUNK // _W
_NP = 10112
_ZR = _NP // _NSUB
_R = 1000
_G = _N // _R


def _agg_body(zflat, src2, dstp, zeros, out, dst_b, sw0, sw1, g0, g1, sp,
              sem0, sem1, ssem0, ssem1, wsem):
    c = lax.axis_index("c")
    s = lax.axis_index("s")
    pltpu.async_copy(dstp.at[s], dst_b, ssem0)
    pltpu.async_copy(src2.at[c, s, pl.ds(0, _W)], sw0, ssem1)
    pltpu.async_copy(zeros, sp.at[pl.ds(s * _ZR, _ZR)], wsem)
    pltpu.make_async_copy(dstp.at[s], dst_b, ssem0).wait()
    pltpu.make_async_copy(src2.at[c, s, pl.ds(0, _W)], sw0, ssem1).wait()
    pltpu.make_async_copy(zeros, sp.at[pl.ds(s * _ZR, _ZR)], wsem).wait()
    plsc.subcore_barrier()

    pltpu.async_copy(zflat.at[sw0.at[0]], g0, sem0)
    pltpu.async_copy(zflat.at[sw0.at[1]], g1, sem1)
    for w in range(_NW):
        sw = (sw0, sw1)[w % 2]
        nx = (sw1, sw0)[w % 2]
        base = w * _W
        last = w == _NW - 1
        if not last:
            wcp = pltpu.async_copy(src2.at[c, s, pl.ds((w + 1) * _W, _W)],
                                   nx, wsem)

        def body(i, carry, sw=sw, base=base):
            j0 = 2 * i
            pltpu.make_async_copy(zflat.at[sw.at[j0]], g0, sem0).wait()
            pltpu.sync_copy(g0, sp.at[dst_b.at[base + j0]], add=True)
            pltpu.async_copy(zflat.at[sw.at[j0 + 2]], g0, sem0)
            pltpu.make_async_copy(zflat.at[sw.at[j0 + 1]], g1, sem1).wait()
            pltpu.sync_copy(g1, sp.at[dst_b.at[base + j0 + 1]], add=True)
            pltpu.async_copy(zflat.at[sw.at[j0 + 3]], g1, sem1)
            return carry

        lax.fori_loop(0, _W // 2 - 1, body, 0)

        j0 = _W - 2
        pltpu.make_async_copy(zflat.at[sw.at[j0]], g0, sem0).wait()
        pltpu.sync_copy(g0, sp.at[dst_b.at[base + j0]], add=True)
        if not last:
            wcp.wait()
            pltpu.async_copy(zflat.at[nx.at[0]], g0, sem0)
        pltpu.make_async_copy(zflat.at[sw.at[j0 + 1]], g1, sem1).wait()
        pltpu.sync_copy(g1, sp.at[dst_b.at[base + j0 + 1]], add=True)
        if not last:
            pltpu.async_copy(zflat.at[nx.at[1]], g1, sem1)
    plsc.subcore_barrier()
    pltpu.sync_copy(sp.at[pl.ds(s * _ZR, _ZR)],
                    out.at[pl.ds(c * _NP + s * _ZR, _ZR)])


@functools.lru_cache(maxsize=1)
def _make_aggregate():
    return functools.partial(
        pl.kernel,
        mesh=plsc.VectorSubcoreMesh(core_axis_name="c", subcore_axis_name="s"),
        out_type=jax.ShapeDtypeStruct((_NCORE * _NP, _HALF), jnp.float32),
        scratch_types=[
            pltpu.VMEM((_NCHUNK, _CH), jnp.int32),
            pltpu.VMEM((_W, _CH), jnp.int32),
            pltpu.VMEM((_W, _CH), jnp.int32),
            pltpu.VMEM((_CH, _HALF), jnp.float32),
            pltpu.VMEM((_CH, _HALF), jnp.float32),
            pltpu.VMEM_SHARED((_NP, _HALF), jnp.float32),
            pltpu.SemaphoreType.DMA,
            pltpu.SemaphoreType.DMA,
            pltpu.SemaphoreType.DMA,
            pltpu.SemaphoreType.DMA,
            pltpu.SemaphoreType.DMA,
        ],
    )(_agg_body)


def _aggregate(zflat, src2, dstp, zeros):
    return _make_aggregate()(zflat, src2, dstp, zeros)


def _mlp_body(z_ref, alo_ref, ahi_ref, w1_ref, b1_ref, w2_ref, b2_ref,
              h_ref, ps_ref, pss_ref):
    i = pl.program_id(0)
    s = z_ref[...] + jnp.concatenate([alo_ref[0], ahi_ref[0]], axis=1)
    dn = (((1,), (1,)), ((), ()))
    h1 = jnp.maximum(
        lax.dot_general(s, w1_ref[...], dn,
                        preferred_element_type=jnp.float32) + b1_ref[...], 0.0)
    h2 = jnp.maximum(
        lax.dot_general(h1, w2_ref[...], dn,
                        preferred_element_type=jnp.float32) + b2_ref[...], 0.0)
    h_ref[...] = h2

    @pl.when(i == 0)
    def _():
        ps_ref[...] = jnp.zeros_like(ps_ref)
        pss_ref[...] = jnp.zeros_like(pss_ref)

    ps_ref[0:1, :] += jnp.sum(h2, axis=0, keepdims=True)
    pss_ref[0:1, :] += jnp.sum(h2 * h2, axis=0, keepdims=True)


def _mlp(z, aggp, w1, b1, w2, b2):
    return pl.pallas_call(
        _mlp_body,
        grid=(_G,),
        in_specs=[
            pl.BlockSpec((_R, _D), lambda b: (b, 0)),
            pl.BlockSpec((1, _R, _HALF), lambda b: (0, b, 0)),
            pl.BlockSpec((1, _R, _HALF), lambda b: (1, b, 0)),
            pl.BlockSpec((_D, _D), lambda b: (0, 0)),
            pl.BlockSpec((1, _D), lambda b: (0, 0)),
            pl.BlockSpec((_D, _D), lambda b: (0, 0)),
            pl.BlockSpec((1, _D), lambda b: (0, 0)),
        ],
        out_specs=[
            pl.BlockSpec((_R, _D), lambda b: (b, 0)),
            pl.BlockSpec((8, _D), lambda b: (0, 0)),
            pl.BlockSpec((8, _D), lambda b: (0, 0)),
        ],
        out_shape=[
            jax.ShapeDtypeStruct((_N, _D), jnp.float32),
            jax.ShapeDtypeStruct((8, _D), jnp.float32),
            jax.ShapeDtypeStruct((8, _D), jnp.float32),
        ],
    )(z, aggp, aggp, w1, b1.reshape(1, _D), w2, b2.reshape(1, _D))


def _norm_body(h_ref, sc_ref, sh_ref, z_ref, zf_ref):
    hh = pl.program_id(1)
    sc = jnp.where(hh == 0, sc_ref[0:1, :], sc_ref[1:2, :])
    sh = jnp.where(hh == 0, sh_ref[0:1, :], sh_ref[1:2, :])
    v = h_ref[...] * sc + sh
    z_ref[...] = v
    zf_ref[0] = v


def _norm(h, scale, shift):
    return pl.pallas_call(
        _norm_body,
        grid=(_G, 2),
        in_specs=[
            pl.BlockSpec((_R, _HALF), lambda b, hh: (b, hh)),
            pl.BlockSpec((2, _HALF), lambda b, hh: (0, 0)),
            pl.BlockSpec((2, _HALF), lambda b, hh: (0, 0)),
        ],
        out_specs=[
            pl.BlockSpec((_R, _HALF), lambda b, hh: (b, hh)),
            pl.BlockSpec((1, _R, _HALF), lambda b, hh: (hh, b, 0)),
        ],
        out_shape=[
            jax.ShapeDtypeStruct((_N, _D), jnp.float32),
            jax.ShapeDtypeStruct((_NCORE, _NP, _HALF), jnp.float32),
        ],
    )(h, scale.reshape(2, _HALF), shift.reshape(2, _HALF))


def _layer(z, zflat, src2, dstp, zeros, w1, b1, w2, b2, gamma, beta):
    aggf = _aggregate(zflat, src2, dstp, zeros)
    h, ps, pss = _mlp(z, aggf.reshape(_NCORE, _NP, _HALF), w1, b1, w2, b2)
    mean = ps[0] / _N
    var = pss[0] / _N - mean * mean
    scale = gamma * lax.rsqrt(var + 1e-5)
    shift = beta - mean * scale
    z, zf = _norm(h, scale, shift)
    return z, zf.reshape(_NCORE * _NP, _HALF)


def _impl(x, edge_index, W1_0, b1_0, W2_0, b2_0, gamma_0, beta_0,
          W1_1, b1_1, W2_1, b2_1, gamma_1, beta_1):
    src = edge_index[0]
    dst = edge_index[1]
    srcr = jnp.pad(src.reshape(_NSUB, _EPT), ((0, 0), (0, _EPAD - _EPT)))
    srcr = srcr.reshape(_NSUB, _NCHUNK, _CH)
    src2 = jnp.stack([srcr, srcr + _NP])
    dstr = jnp.pad(dst.reshape(_NSUB, _EPT), ((0, 0), (0, _EPAD - _EPT)),
                   constant_values=_N)
    dstp = dstr.reshape(_NSUB, _NCHUNK, _CH)
    zeros = jnp.zeros((_ZR, _HALF), jnp.float32)

    xp = jnp.pad(x.reshape(_N, 2, _HALF).transpose(1, 0, 2),
                 ((0, 0), (0, _NP - _N), (0, 0)))
    xflat = xp.reshape(_NCORE * _NP, _HALF)
    z1, z1flat = _layer(x, xflat, src2, dstp, zeros,
                        W1_0, b1_0, W2_0, b2_0, gamma_0, beta_0)
    z2, _ = _layer(z1, z1flat, src2, dstp, zeros,
                   W1_1, b1_1, W2_1, b2_1, gamma_1, beta_1)
    return jnp.concatenate([z1, z2], axis=1)


kernel = jax.jit(_impl)


# speedup vs baseline: 1.1365x; 1.0126x over previous
import functools

import jax
import jax.numpy as jnp
from jax import lax
from jax.experimental import pallas as pl
from jax.experimental.pallas import tpu as pltpu
from jax.experimental.pallas import tpu_sc as plsc

_N = 10000
_E = 160000
_D = 256
_HALF = 128
_NSUB = 16
_NCORE = 2
_EPT = _E // _NSUB
_CH = 128
_NCHUNK = 80
_EPAD = _NCHUNK * _CH
_W = 16
_NW = _NCHUNK // _W
_NP = 10112
_ZR = _NP // _NSUB
_R = 1000
_G = _N // _R


def _agg_body(zflat, src2, dstp, zeros, out, dst_b, sw0, sw1, g0, g1, sp,
              sem0, sem1, ssem0, ssem1, wsem):
    c = lax.axis_index("c")
    s = lax.axis_index("s")
    pltpu.async_copy(dstp.at[s], dst_b, ssem0)
    pltpu.async_copy(src2.at[c, s, pl.ds(0, _W)], sw0, ssem1)
    pltpu.async_copy(zeros, sp.at[pl.ds(s * _ZR, _ZR)], wsem)
    pltpu.make_async_copy(dstp.at[s], dst_b, ssem0).wait()
    pltpu.make_async_copy(src2.at[c, s, pl.ds(0, _W)], sw0, ssem1).wait()
    pltpu.make_async_copy(zeros, sp.at[pl.ds(s * _ZR, _ZR)], wsem).wait()
    plsc.subcore_barrier()

    pltpu.async_copy(zflat.at[sw0.at[0]], g0, sem0)
    pltpu.async_copy(zflat.at[sw0.at[1]], g1, sem1)
    for w in range(_NW):
        sw = (sw0, sw1)[w % 2]
        nx = (sw1, sw0)[w % 2]
        base = w * _W
        last = w == _NW - 1
        if not last:
            wcp = pltpu.async_copy(src2.at[c, s, pl.ds((w + 1) * _W, _W)],
                                   nx, wsem)

        def body(i, carry, sw=sw, base=base):
            j0 = 2 * i
            pltpu.make_async_copy(zflat.at[sw.at[j0]], g0, sem0).wait()
            pltpu.sync_copy(g0, sp.at[dst_b.at[base + j0]], add=True)
            pltpu.async_copy(zflat.at[sw.at[j0 + 2]], g0, sem0)
            pltpu.make_async_copy(zflat.at[sw.at[j0 + 1]], g1, sem1).wait()
            pltpu.sync_copy(g1, sp.at[dst_b.at[base + j0 + 1]], add=True)
            pltpu.async_copy(zflat.at[sw.at[j0 + 3]], g1, sem1)
            return carry

        lax.fori_loop(0, _W // 2 - 1, body, 0)

        j0 = _W - 2
        pltpu.make_async_copy(zflat.at[sw.at[j0]], g0, sem0).wait()
        pltpu.sync_copy(g0, sp.at[dst_b.at[base + j0]], add=True)
        if not last:
            wcp.wait()
            pltpu.async_copy(zflat.at[nx.at[0]], g0, sem0)
        pltpu.make_async_copy(zflat.at[sw.at[j0 + 1]], g1, sem1).wait()
        pltpu.sync_copy(g1, sp.at[dst_b.at[base + j0 + 1]], add=True)
        if not last:
            pltpu.async_copy(zflat.at[nx.at[1]], g1, sem1)
    plsc.subcore_barrier()
    pltpu.sync_copy(sp.at[pl.ds(s * _ZR, _ZR)],
                    out.at[pl.ds(c * _NP + s * _ZR, _ZR)])


@functools.lru_cache(maxsize=1)
def _make_aggregate():
    return functools.partial(
        pl.kernel,
        mesh=plsc.VectorSubcoreMesh(core_axis_name="c", subcore_axis_name="s"),
        out_type=jax.ShapeDtypeStruct((_NCORE * _NP, _HALF), jnp.float32),
        scratch_types=[
            pltpu.VMEM((_NCHUNK, _CH), jnp.int32),
            pltpu.VMEM((_W, _CH), jnp.int32),
            pltpu.VMEM((_W, _CH), jnp.int32),
            pltpu.VMEM((_CH, _HALF), jnp.float32),
            pltpu.VMEM((_CH, _HALF), jnp.float32),
            pltpu.VMEM_SHARED((_NP, _HALF), jnp.float32),
            pltpu.SemaphoreType.DMA,
            pltpu.SemaphoreType.DMA,
            pltpu.SemaphoreType.DMA,
            pltpu.SemaphoreType.DMA,
            pltpu.SemaphoreType.DMA,
        ],
    )(_agg_body)


def _aggregate(zflat, src2, dstp, zeros):
    return _make_aggregate()(zflat, src2, dstp, zeros)


def _mlp_body(z_ref, alo_ref, ahi_ref, w1_ref, b1_ref, w2_ref, b2_ref,
              h_ref, ps_ref, pss_ref):
    i = pl.program_id(0)
    s = z_ref[...] + jnp.concatenate([alo_ref[0], ahi_ref[0]], axis=1)
    dn = (((1,), (1,)), ((), ()))
    h1 = jnp.maximum(
        lax.dot_general(s, w1_ref[...], dn,
                        preferred_element_type=jnp.float32) + b1_ref[...], 0.0)
    h2 = jnp.maximum(
        lax.dot_general(h1, w2_ref[...], dn,
                        preferred_element_type=jnp.float32) + b2_ref[...], 0.0)
    h_ref[...] = h2

    @pl.when(i == 0)
    def _():
        ps_ref[...] = jnp.zeros_like(ps_ref)
        pss_ref[...] = jnp.zeros_like(pss_ref)

    ps_ref[0:1, :] += jnp.sum(h2, axis=0, keepdims=True)
    pss_ref[0:1, :] += jnp.sum(h2 * h2, axis=0, keepdims=True)


def _mlp(z, aggp, w1, b1, w2, b2):
    return pl.pallas_call(
        _mlp_body,
        grid=(_G,),
        in_specs=[
            pl.BlockSpec((_R, _D), lambda b: (b, 0)),
            pl.BlockSpec((1, _R, _HALF), lambda b: (0, b, 0)),
            pl.BlockSpec((1, _R, _HALF), lambda b: (1, b, 0)),
            pl.BlockSpec((_D, _D), lambda b: (0, 0)),
            pl.BlockSpec((1, _D), lambda b: (0, 0)),
            pl.BlockSpec((_D, _D), lambda b: (0, 0)),
            pl.BlockSpec((1, _D), lambda b: (0, 0)),
        ],
        out_specs=[
            pl.BlockSpec((_R, _D), lambda b: (b, 0)),
            pl.BlockSpec((8, _D), lambda b: (0, 0)),
            pl.BlockSpec((8, _D), lambda b: (0, 0)),
        ],
        out_shape=[
            jax.ShapeDtypeStruct((_N, _D), jnp.float32),
            jax.ShapeDtypeStruct((8, _D), jnp.float32),
            jax.ShapeDtypeStruct((8, _D), jnp.float32),
        ],
    )(z, aggp, aggp, w1, b1.reshape(1, _D), w2, b2.reshape(1, _D))


def _norm_body(h_ref, sc_ref, sh_ref, buf_ref, z_ref, zf_ref):
    hh = pl.program_id(1)
    sc = jnp.where(hh == 0, sc_ref[0:1, :], sc_ref[1:2, :])
    sh = jnp.where(hh == 0, sh_ref[0:1, :], sh_ref[1:2, :])
    v = h_ref[...] * sc + sh
    z_ref[...] = v
    zf_ref[0] = v


def _norm(h, scale, shift, buf, lay):
    return pl.pallas_call(
        _norm_body,
        grid=(_G, 2),
        in_specs=[
            pl.BlockSpec((_R, _HALF), lambda b, hh: (b, hh)),
            pl.BlockSpec((2, _HALF), lambda b, hh: (0, 0)),
            pl.BlockSpec((2, _HALF), lambda b, hh: (0, 0)),
            pl.BlockSpec(memory_space=pltpu.HBM),
        ],
        out_specs=[
            pl.BlockSpec((_R, _HALF), lambda b, hh, lay=lay: (b, 2 * lay + hh)),
            pl.BlockSpec((1, _R, _HALF), lambda b, hh: (hh, b, 0)),
        ],
        out_shape=[
            jax.ShapeDtypeStruct((_N, 2 * _D), jnp.float32),
            jax.ShapeDtypeStruct((_NCORE, _NP, _HALF), jnp.float32),
        ],
        input_output_aliases={3: 0},
    )(h, scale.reshape(2, _HALF), shift.reshape(2, _HALF), buf)


def _layer(z, zflat, src2, dstp, zeros, w1, b1, w2, b2, gamma, beta, buf,
           lay):
    aggf = _aggregate(zflat, src2, dstp, zeros)
    h, ps, pss = _mlp(z, aggf.reshape(_NCORE, _NP, _HALF), w1, b1, w2, b2)
    mean = ps[0] / _N
    var = pss[0] / _N - mean * mean
    scale = gamma * lax.rsqrt(var + 1e-5)
    shift = beta - mean * scale
    z, zf = _norm(h, scale, shift, buf, lay)
    return z, zf.reshape(_NCORE * _NP, _HALF)


def _impl(x, edge_index, W1_0, b1_0, W2_0, b2_0, gamma_0, beta_0,
          W1_1, b1_1, W2_1, b2_1, gamma_1, beta_1):
    src = edge_index[0]
    dst = edge_index[1]
    srcr = jnp.pad(src.reshape(_NSUB, _EPT), ((0, 0), (0, _EPAD - _EPT)))
    srcr = srcr.reshape(_NSUB, _NCHUNK, _CH)
    src2 = jnp.stack([srcr, srcr + _NP])
    dstr = jnp.pad(dst.reshape(_NSUB, _EPT), ((0, 0), (0, _EPAD - _EPT)),
                   constant_values=_N)
    dstp = dstr.reshape(_NSUB, _NCHUNK, _CH)
    zeros = jnp.zeros((_ZR, _HALF), jnp.float32)

    xp = jnp.pad(x.reshape(_N, 2, _HALF).transpose(1, 0, 2),
                 ((0, 0), (0, _NP - _N), (0, 0)))
    xflat = xp.reshape(_NCORE * _NP, _HALF)
    buf = jnp.zeros((_N, 2 * _D), jnp.float32)
    z1, z1flat = _layer(x, xflat, src2, dstp, zeros,
                        W1_0, b1_0, W2_0, b2_0, gamma_0, beta_0, buf, 0)
    z2, _ = _layer(z1, z1flat, src2, dstp, zeros,
                   W1_1, b1_1, W2_1, b2_1, gamma_1, beta_1, z1, 1)
    return z2


kernel = jax.jit(_impl)


# speedup vs baseline: 1.1376x; 1.0010x over previous
import functools

import jax
import jax.numpy as jnp
from jax import lax
from jax.experimental import pallas as pl
from jax.experimental.pallas import tpu as pltpu
from jax.experimental.pallas import tpu_sc as plsc

_N = 10000
_E = 160000
_D = 256
_HALF = 128
_NSUB = 16
_NCORE = 2
_EPT = _E // _NSUB
_CH = 128
_NCHUNK = 80
_EPAD = _NCHUNK * _CH
_W = 16
_NW = _NCHUNK // _W
_NP = 10112
_ZR = _NP // _NSUB
_R = 1000
_G = _N // _R


def _agg_body(zflat, src2, dstp, zeros, out, dst_b, sw0, sw1, g0, g1, sp,
              sem0, sem1, ssem0, ssem1, wsem):
    c = lax.axis_index("c")
    s = lax.axis_index("s")
    pltpu.async_copy(dstp.at[s], dst_b, ssem0)
    pltpu.async_copy(src2.at[c, s, pl.ds(0, _W)], sw0, ssem1)
    pltpu.async_copy(zeros, sp.at[pl.ds(s * _ZR, _ZR)], wsem)
    pltpu.make_async_copy(dstp.at[s], dst_b, ssem0).wait()
    pltpu.make_async_copy(src2.at[c, s, pl.ds(0, _W)], sw0, ssem1).wait()
    pltpu.make_async_copy(zeros, sp.at[pl.ds(s * _ZR, _ZR)], wsem).wait()
    plsc.subcore_barrier()

    pltpu.async_copy(zflat.at[sw0.at[0]], g0, sem0)
    pltpu.async_copy(zflat.at[sw0.at[1]], g1, sem1)
    for w in range(_NW):
        sw = (sw0, sw1)[w % 2]
        nx = (sw1, sw0)[w % 2]
        base = w * _W
        last = w == _NW - 1
        if not last:
            wcp = pltpu.async_copy(src2.at[c, s, pl.ds((w + 1) * _W, _W)],
                                   nx, wsem)

        def body(i, carry, sw=sw, base=base):
            j0 = 2 * i
            pltpu.make_async_copy(zflat.at[sw.at[j0]], g0, sem0).wait()
            pltpu.sync_copy(g0, sp.at[dst_b.at[base + j0]], add=True)
            pltpu.async_copy(zflat.at[sw.at[j0 + 2]], g0, sem0)
            pltpu.make_async_copy(zflat.at[sw.at[j0 + 1]], g1, sem1).wait()
            pltpu.sync_copy(g1, sp.at[dst_b.at[base + j0 + 1]], add=True)
            pltpu.async_copy(zflat.at[sw.at[j0 + 3]], g1, sem1)
            return carry

        lax.fori_loop(0, _W // 2 - 1, body, 0)

        j0 = _W - 2
        pltpu.make_async_copy(zflat.at[sw.at[j0]], g0, sem0).wait()
        pltpu.sync_copy(g0, sp.at[dst_b.at[base + j0]], add=True)
        if not last:
            wcp.wait()
            pltpu.async_copy(zflat.at[nx.at[0]], g0, sem0)
        pltpu.make_async_copy(zflat.at[sw.at[j0 + 1]], g1, sem1).wait()
        pltpu.sync_copy(g1, sp.at[dst_b.at[base + j0 + 1]], add=True)
        if not last:
            pltpu.async_copy(zflat.at[nx.at[1]], g1, sem1)
    plsc.subcore_barrier()
    pltpu.sync_copy(sp.at[pl.ds(s * _ZR, _ZR)],
                    out.at[pl.ds(c * _NP + s * _ZR, _ZR)])


@functools.lru_cache(maxsize=1)
def _make_aggregate():
    return functools.partial(
        pl.kernel,
        mesh=plsc.VectorSubcoreMesh(core_axis_name="c", subcore_axis_name="s"),
        out_type=jax.ShapeDtypeStruct((_NCORE * _NP, _HALF), jnp.float32),
        scratch_types=[
            pltpu.VMEM((_NCHUNK, _CH), jnp.int32),
            pltpu.VMEM((_W, _CH), jnp.int32),
            pltpu.VMEM((_W, _CH), jnp.int32),
            pltpu.VMEM((_CH, _HALF), jnp.float32),
            pltpu.VMEM((_CH, _HALF), jnp.float32),
            pltpu.VMEM_SHARED((_NP, _HALF), jnp.float32),
            pltpu.SemaphoreType.DMA,
            pltpu.SemaphoreType.DMA,
            pltpu.SemaphoreType.DMA,
            pltpu.SemaphoreType.DMA,
            pltpu.SemaphoreType.DMA,
        ],
    )(_agg_body)


def _aggregate(zflat, src2, dstp, zeros):
    return _make_aggregate()(zflat, src2, dstp, zeros)


def _mlp_body(z_ref, alo_ref, ahi_ref, w1_ref, b1_ref, w2_ref, b2_ref,
              h_ref, ps_ref, pss_ref):
    i = pl.program_id(0)
    s = z_ref[...] + jnp.concatenate([alo_ref[0], ahi_ref[0]], axis=1)
    dn = (((1,), (1,)), ((), ()))
    h1 = jnp.maximum(
        lax.dot_general(s.astype(jnp.bfloat16), w1_ref[...].astype(jnp.bfloat16),
                        dn, preferred_element_type=jnp.float32)
        + b1_ref[...], 0.0)
    h2 = jnp.maximum(
        lax.dot_general(h1.astype(jnp.bfloat16), w2_ref[...].astype(jnp.bfloat16),
                        dn, preferred_element_type=jnp.float32)
        + b2_ref[...], 0.0)
    h_ref[...] = h2

    @pl.when(i == 0)
    def _():
        ps_ref[...] = jnp.zeros_like(ps_ref)
        pss_ref[...] = jnp.zeros_like(pss_ref)

    ps_ref[0:1, :] += jnp.sum(h2, axis=0, keepdims=True)
    pss_ref[0:1, :] += jnp.sum(h2 * h2, axis=0, keepdims=True)


def _mlp(z, aggp, w1, b1, w2, b2):
    return pl.pallas_call(
        _mlp_body,
        grid=(_G,),
        in_specs=[
            pl.BlockSpec((_R, _D), lambda b: (b, 0)),
            pl.BlockSpec((1, _R, _HALF), lambda b: (0, b, 0)),
            pl.BlockSpec((1, _R, _HALF), lambda b: (1, b, 0)),
            pl.BlockSpec((_D, _D), lambda b: (0, 0)),
            pl.BlockSpec((1, _D), lambda b: (0, 0)),
            pl.BlockSpec((_D, _D), lambda b: (0, 0)),
            pl.BlockSpec((1, _D), lambda b: (0, 0)),
        ],
        out_specs=[
            pl.BlockSpec((_R, _D), lambda b: (b, 0)),
            pl.BlockSpec((8, _D), lambda b: (0, 0)),
            pl.BlockSpec((8, _D), lambda b: (0, 0)),
        ],
        out_shape=[
            jax.ShapeDtypeStruct((_N, _D), jnp.float32),
            jax.ShapeDtypeStruct((8, _D), jnp.float32),
            jax.ShapeDtypeStruct((8, _D), jnp.float32),
        ],
    )(z, aggp, aggp, w1, b1.reshape(1, _D), w2, b2.reshape(1, _D))


def _norm_body(h_ref, sc_ref, sh_ref, buf_ref, z_ref, zf_ref):
    hh = pl.program_id(1)
    sc = jnp.where(hh == 0, sc_ref[0:1, :], sc_ref[1:2, :])
    sh = jnp.where(hh == 0, sh_ref[0:1, :], sh_ref[1:2, :])
    v = h_ref[...] * sc + sh
    z_ref[...] = v
    zf_ref[0] = v


def _norm(h, scale, shift, buf, lay):
    return pl.pallas_call(
        _norm_body,
        grid=(_G, 2),
        in_specs=[
            pl.BlockSpec((_R, _HALF), lambda b, hh: (b, hh)),
            pl.BlockSpec((2, _HALF), lambda b, hh: (0, 0)),
            pl.BlockSpec((2, _HALF), lambda b, hh: (0, 0)),
            pl.BlockSpec(memory_space=pltpu.HBM),
        ],
        out_specs=[
            pl.BlockSpec((_R, _HALF), lambda b, hh, lay=lay: (b, 2 * lay + hh)),
            pl.BlockSpec((1, _R, _HALF), lambda b, hh: (hh, b, 0)),
        ],
        out_shape=[
            jax.ShapeDtypeStruct((_N, 2 * _D), jnp.float32),
            jax.ShapeDtypeStruct((_NCORE, _NP, _HALF), jnp.float32),
        ],
        input_output_aliases={3: 0},
    )(h, scale.reshape(2, _HALF), shift.reshape(2, _HALF), buf)


def _layer(z, zflat, src2, dstp, zeros, w1, b1, w2, b2, gamma, beta, buf,
           lay):
    aggf = _aggregate(zflat, src2, dstp, zeros)
    h, ps, pss = _mlp(z, aggf.reshape(_NCORE, _NP, _HALF), w1, b1, w2, b2)
    mean = ps[0] / _N
    var = pss[0] / _N - mean * mean
    scale = gamma * lax.rsqrt(var + 1e-5)
    shift = beta - mean * scale
    z, zf = _norm(h, scale, shift, buf, lay)
    return z, zf.reshape(_NCORE * _NP, _HALF)


def _impl(x, edge_index, W1_0, b1_0, W2_0, b2_0, gamma_0, beta_0,
          W1_1, b1_1, W2_1, b2_1, gamma_1, beta_1):
    src = edge_index[0]
    dst = edge_index[1]
    srcr = jnp.pad(src.reshape(_NSUB, _EPT), ((0, 0), (0, _EPAD - _EPT)))
    srcr = srcr.reshape(_NSUB, _NCHUNK, _CH)
    src2 = jnp.stack([srcr, srcr + _NP])
    dstr = jnp.pad(dst.reshape(_NSUB, _EPT), ((0, 0), (0, _EPAD - _EPT)),
                   constant_values=_N)
    dstp = dstr.reshape(_NSUB, _NCHUNK, _CH)
    zeros = jnp.zeros((_ZR, _HALF), jnp.float32)

    xp = jnp.pad(x.reshape(_N, 2, _HALF).transpose(1, 0, 2),
                 ((0, 0), (0, _NP - _N), (0, 0)))
    xflat = xp.reshape(_NCORE * _NP, _HALF)
    buf = jnp.zeros((_N, 2 * _D), jnp.float32)
    z1, z1flat = _layer(x, xflat, src2, dstp, zeros,
                        W1_0, b1_0, W2_0, b2_0, gamma_0, beta_0, buf, 0)
    z2, _ = _layer(z1, z1flat, src2, dstp, zeros,
                   W1_1, b1_1, W2_1, b2_1, gamma_1, beta_1, z1, 1)
    return z2


kernel = jax.jit(_impl)
